# Initial kernel scaffold; baseline (speedup 1.0000x reference)
import jax
import jax.numpy as jnp
from jax.experimental import pallas as pl


def kernel(x, Wg, W1, b1, W2, b2):
    raise NotImplementedError("write your pallas kernel here")


# speedup vs baseline: 1.3023x; 1.3023x over previous
import functools

import jax
import jax.numpy as jnp
from jax.experimental import pallas as pl
from jax.experimental.pallas import tpu as pltpu

B, S, DIM = 1, 2048, 768
E, TOPK, HID = 8, 2, 2048

BT = 512


def _router_cw(xb, wg):
    logits = jnp.dot(xb, wg, preferred_element_type=jnp.float32)
    m = jnp.max(logits, axis=-1, keepdims=True)
    p = jnp.exp(logits - m)
    p = p / jnp.sum(p, axis=-1, keepdims=True)
    idx = jax.lax.broadcasted_iota(jnp.int32, p.shape, 1)
    m1 = jnp.max(p, axis=-1, keepdims=True)
    i1 = jnp.min(jnp.where(p == m1, idx, E), axis=-1, keepdims=True)
    p2 = jnp.where(idx == i1, -jnp.inf, p)
    m2 = jnp.max(p2, axis=-1, keepdims=True)
    i2 = jnp.min(jnp.where(p2 == m2, idx, E), axis=-1, keepdims=True)
    denom = m1 + m2 + 1e-9
    w1 = m1 / denom
    w2 = m2 / denom
    return jnp.where(idx == i1, w1, 0.0) + jnp.where(idx == i2, w2, 0.0)


def _moe_body(x_ref, wg_ref, w1_ref, b1_ref, w2_ref, b2_ref, y_ref, cw_ref):
    e = pl.program_id(1)
    xb = x_ref[0]

    @pl.when(e == 0)
    def _():
        cw_ref[...] = _router_cw(xb, wg_ref[...])

    h = jnp.maximum(
        jnp.dot(xb, w1_ref[0], preferred_element_type=jnp.float32) + b1_ref[0],
        0.0,
    )
    o = jnp.dot(h, w2_ref[0], preferred_element_type=jnp.float32) + b2_ref[0]
    cw = cw_ref[...]
    eidx = jax.lax.broadcasted_iota(jnp.int32, cw.shape, 1)
    cw_e = jnp.sum(jnp.where(eidx == e, cw, 0.0), axis=1, keepdims=True)
    contrib = cw_e * o

    @pl.when(e == 0)
    def _():
        y_ref[0] = contrib

    @pl.when(e != 0)
    def _():
        y_ref[0] = y_ref[0] + contrib


@jax.jit
def kernel(x, Wg, W1, b1, W2, b2):
    grid = (S // BT, E)
    y = pl.pallas_call(
        _moe_body,
        grid=grid,
        in_specs=[
            pl.BlockSpec((1, BT, DIM), lambda i, e: (0, i, 0)),
            pl.BlockSpec((DIM, E), lambda i, e: (0, 0)),
            pl.BlockSpec((1, DIM, HID), lambda i, e: (e, 0, 0)),
            pl.BlockSpec((1, 1, HID), lambda i, e: (e, 0, 0)),
            pl.BlockSpec((1, HID, DIM), lambda i, e: (e, 0, 0)),
            pl.BlockSpec((1, 1, DIM), lambda i, e: (e, 0, 0)),
        ],
        out_specs=pl.BlockSpec((1, BT, DIM), lambda i, e: (0, i, 0)),
        out_shape=jax.ShapeDtypeStruct((B, S, DIM), jnp.float32),
        scratch_shapes=[pltpu.VMEM((BT, E), jnp.float32)],
        compiler_params=pltpu.CompilerParams(
            dimension_semantics=("parallel", "arbitrary"),
        ),
    )(x, Wg, W1, b1.reshape(E, 1, HID), W2, b2.reshape(E, 1, DIM))
    return y
